# baseline (speedup 1.0000x reference)
import functools

import jax
import jax.numpy as jnp
from jax import lax
from jax.experimental import pallas as pl
from jax.experimental.pallas import tpu as pltpu
from jax.experimental.pallas import tpu_sc as plsc

B = 16384
D = 32
NC = 2
NS = 16
NW = NC * NS
BPW = B // NW
CH = 128
NCH = BPW // CH


def _glove_lookup(i, j, W, b):
    mesh = plsc.VectorSubcoreMesh(core_axis_name="c", subcore_axis_name="s")

    @functools.partial(
        pl.kernel,
        mesh=mesh,
        compiler_params=pltpu.CompilerParams(use_tc_tiling_on_sc=False),
        out_type=(
            jax.ShapeDtypeStruct((B, D), jnp.float32),
            jax.ShapeDtypeStruct((B, D), jnp.float32),
            jax.ShapeDtypeStruct((B,), jnp.float32),
            jax.ShapeDtypeStruct((B,), jnp.float32),
        ),
        scratch_types=[
            pltpu.VMEM((BPW,), jnp.int32),
            pltpu.VMEM((BPW,), jnp.int32),
            pltpu.VMEM((BPW, D), jnp.float32),
            pltpu.VMEM((BPW, D), jnp.float32),
            pltpu.VMEM((BPW,), jnp.float32),
            pltpu.VMEM((BPW,), jnp.float32),
            pltpu.SemaphoreType.DMA,
        ],
    )
    def k(i_hbm, j_hbm, w_hbm, b_hbm,
          wi_hbm, wj_hbm, bi_hbm, bj_hbm,
          idx_i, idx_j, rows_i, rows_j, bv_i, bv_j, sem):
        wid = lax.axis_index("s") * NC + lax.axis_index("c")
        base = wid * BPW
        pltpu.sync_copy(i_hbm.at[pl.ds(base, BPW)], idx_i)
        pltpu.sync_copy(j_hbm.at[pl.ds(base, BPW)], idx_j)
        copies = []
        for c in range(NCH):
            sl = pl.ds(c * CH, CH)
            copies.append(pltpu.async_copy(w_hbm.at[idx_i.at[sl]], rows_i.at[sl], sem))
            copies.append(pltpu.async_copy(w_hbm.at[idx_j.at[sl]], rows_j.at[sl], sem))
            copies.append(pltpu.async_copy(b_hbm.at[idx_i.at[sl]], bv_i.at[sl], sem))
            copies.append(pltpu.async_copy(b_hbm.at[idx_j.at[sl]], bv_j.at[sl], sem))
        for cp in copies:
            cp.wait()
        out_sl = pl.ds(base, BPW)
        pltpu.sync_copy(rows_i, wi_hbm.at[out_sl])
        pltpu.sync_copy(rows_j, wj_hbm.at[out_sl])
        pltpu.sync_copy(bv_i, bi_hbm.at[out_sl])
        pltpu.sync_copy(bv_j, bj_hbm.at[out_sl])

    return k(i, j, W, b)


def kernel(i, j, W, b):
    return _glove_lookup(i, j, W, b)
